# baseline (device time: 78039 ns/iter reference)
import jax
import jax.numpy as jnp
from jax import lax
from jax.experimental import pallas as pl
from jax.experimental.pallas import tpu as pltpu

N_DEV = 32
N_TOK = 512
N_EXP = 64
D_IN = 256
D_OUT = 512
E_LOCAL = 2
CAP = 6
SLOTS = E_LOCAL * CAP
N_SLOTS = N_DEV * SLOTS


def kernel(x, router_W, route_idx, expert_W):
    del router_W

    def body(x_ref, idx_ref, w_ref, out_ref, comm_ref, send_sems, recv_sems):
        my = lax.axis_index("i")
        left = lax.rem(my + N_DEV - 1, N_DEV)
        right = lax.rem(my + 1, N_DEV)

        barrier = pltpu.get_barrier_semaphore()
        for nbr in (left, right):
            pl.semaphore_signal(
                barrier, inc=1,
                device_id=(nbr,), device_id_type=pl.DeviceIdType.MESH,
            )
        pl.semaphore_wait(barrier, 2)

        e = idx_ref[:, :]
        oh = (e == lax.broadcasted_iota(jnp.int32, (N_TOK, N_EXP), 1)
              ).astype(jnp.float32)
        r = lax.broadcasted_iota(jnp.int32, (N_TOK, N_TOK), 0)
        c = lax.broadcasted_iota(jnp.int32, (N_TOK, N_TOK), 1)
        tri = (r > c).astype(jnp.float32)
        prior = jnp.dot(tri, oh, preferred_element_type=jnp.float32)
        pos = jnp.sum(prior * oh, axis=1, keepdims=True).astype(jnp.int32)
        keep = pos < CAP

        owner = lax.div(e, E_LOCAL)
        le = lax.rem(e, E_LOCAL)
        mine = jnp.logical_and(owner == my, keep)
        s_col = jnp.where(mine, le * CAP + pos, -1)
        st = (s_col == lax.broadcasted_iota(jnp.int32, (N_TOK, SLOTS), 1)
              ).astype(jnp.float32)
        xs = lax.dot_general(st, x_ref[:, :], (((0,), (0,)), ((), ())),
                             preferred_element_type=jnp.float32)
        y0 = jnp.dot(xs, w_ref[0], preferred_element_type=jnp.float32)
        y1 = jnp.dot(xs, w_ref[1], preferred_element_type=jnp.float32)
        row = lax.broadcasted_iota(jnp.int32, (SLOTS, D_OUT), 0)
        comm_ref[0, :, :] = jnp.where(row < CAP, y0, y1)

        for h in range(N_DEV - 1):
            rdma = pltpu.make_async_remote_copy(
                src_ref=comm_ref.at[h],
                dst_ref=comm_ref.at[h + 1],
                send_sem=send_sems.at[h],
                recv_sem=recv_sems.at[h],
                device_id=(right,),
                device_id_type=pl.DeviceIdType.MESH,
            )
            rdma.start()
            rdma.wait()

        k = lax.rem(my - owner + N_DEV, N_DEV)
        g_col = jnp.where(keep, k * SLOTS + le * CAP + pos, -1)
        p = (g_col == lax.broadcasted_iota(jnp.int32, (N_TOK, N_SLOTS), 1)
             ).astype(jnp.float32)
        g = comm_ref[:, :, :].reshape(N_SLOTS, D_OUT)
        out_ref[:, :] = jnp.dot(p, g, preferred_element_type=jnp.float32)

    return pl.pallas_call(
        body,
        out_shape=jax.ShapeDtypeStruct((N_TOK, D_OUT), jnp.float32),
        in_specs=[pl.BlockSpec(memory_space=pltpu.VMEM)] * 3,
        out_specs=pl.BlockSpec(memory_space=pltpu.VMEM),
        scratch_shapes=[
            pltpu.VMEM((N_DEV, SLOTS, D_OUT), jnp.float32),
            pltpu.SemaphoreType.DMA((N_DEV - 1,)),
            pltpu.SemaphoreType.DMA((N_DEV - 1,)),
        ],
        compiler_params=pltpu.CompilerParams(collective_id=0),
    )(x, route_idx, expert_W)


# device time: 26794 ns/iter; 2.9126x vs baseline; 2.9126x over previous
import jax
import jax.numpy as jnp
from jax import lax
from jax.experimental import pallas as pl
from jax.experimental.pallas import tpu as pltpu

N_DEV = 32
N_TOK = 512
N_EXP = 64
D_IN = 256
D_OUT = 512
E_LOCAL = 2
CAP = 6
SLOTS = E_LOCAL * CAP
N_SLOTS = N_DEV * SLOTS


def kernel(x, router_W, route_idx, expert_W):
    del router_W

    def body(x_ref, idx_ref, w_ref, out_ref, comm_ref, send_sems, recv_sems):
        my = lax.axis_index("i")

        barrier = pltpu.get_barrier_semaphore()
        for d in range(1, N_DEV):
            pl.semaphore_signal(
                barrier, inc=1,
                device_id=(lax.rem(my + d, N_DEV),),
                device_id_type=pl.DeviceIdType.MESH,
            )
        pl.semaphore_wait(barrier, N_DEV - 1)

        e = idx_ref[:, :]
        oh = (e == lax.broadcasted_iota(jnp.int32, (N_TOK, N_EXP), 1)
              ).astype(jnp.float32)
        r = lax.broadcasted_iota(jnp.int32, (N_TOK, N_TOK), 0)
        c = lax.broadcasted_iota(jnp.int32, (N_TOK, N_TOK), 1)
        tri = (r > c).astype(jnp.float32)
        prior = jnp.dot(tri, oh, preferred_element_type=jnp.float32)
        pos = jnp.sum(prior * oh, axis=1, keepdims=True).astype(jnp.int32)
        keep = pos < CAP

        owner = lax.div(e, E_LOCAL)
        le = lax.rem(e, E_LOCAL)
        mine = jnp.logical_and(owner == my, keep)
        s_col = jnp.where(mine, le * CAP + pos, -1)
        st = (s_col == lax.broadcasted_iota(jnp.int32, (N_TOK, SLOTS), 1)
              ).astype(jnp.float32)
        xs = lax.dot_general(st, x_ref[:, :], (((0,), (0,)), ((), ())),
                             preferred_element_type=jnp.float32)
        y0 = jnp.dot(xs, w_ref[0], preferred_element_type=jnp.float32)
        y1 = jnp.dot(xs, w_ref[1], preferred_element_type=jnp.float32)
        row = lax.broadcasted_iota(jnp.int32, (SLOTS, D_OUT), 0)
        comm_ref[0, :, :] = jnp.where(row < CAP, y0, y1)

        rdmas = []
        for d in range(1, N_DEV):
            rdma = pltpu.make_async_remote_copy(
                src_ref=comm_ref.at[0],
                dst_ref=comm_ref.at[d],
                send_sem=send_sems.at[d - 1],
                recv_sem=recv_sems.at[d - 1],
                device_id=(lax.rem(my + d, N_DEV),),
                device_id_type=pl.DeviceIdType.MESH,
            )
            rdma.start()
            rdmas.append(rdma)

        k = lax.rem(my - owner + N_DEV, N_DEV)
        g_col = jnp.where(keep, k * SLOTS + le * CAP + pos, -1)
        p = (g_col == lax.broadcasted_iota(jnp.int32, (N_TOK, N_SLOTS), 1)
             ).astype(jnp.float32)

        for rdma in rdmas:
            rdma.wait_recv()

        g = comm_ref[:, :, :].reshape(N_SLOTS, D_OUT)
        out_ref[:, :] = jnp.dot(p, g, preferred_element_type=jnp.float32)

        for rdma in rdmas:
            rdma.wait_send()

    return pl.pallas_call(
        body,
        out_shape=jax.ShapeDtypeStruct((N_TOK, D_OUT), jnp.float32),
        in_specs=[pl.BlockSpec(memory_space=pltpu.VMEM)] * 3,
        out_specs=pl.BlockSpec(memory_space=pltpu.VMEM),
        scratch_shapes=[
            pltpu.VMEM((N_DEV, SLOTS, D_OUT), jnp.float32),
            pltpu.SemaphoreType.DMA((N_DEV - 1,)),
            pltpu.SemaphoreType.DMA((N_DEV - 1,)),
        ],
        compiler_params=pltpu.CompilerParams(collective_id=0),
    )(x, route_idx, expert_W)


# device time: 21460 ns/iter; 3.6365x vs baseline; 1.2486x over previous
import jax
import jax.numpy as jnp
from jax import lax
from jax.experimental import pallas as pl
from jax.experimental.pallas import tpu as pltpu

N_DEV = 32
N_TOK = 512
N_EXP = 64
D_IN = 256
D_OUT = 512
E_LOCAL = 2
CAP = 6
SLOTS = E_LOCAL * CAP
N_SLOTS = N_DEV * SLOTS


def kernel(x, router_W, route_idx, expert_W):
    del router_W

    def body(x_ref, idx_ref, w_ref, out_ref, comm_ref, send_sems, recv_sems):
        my = lax.axis_index("i")

        barrier = pltpu.get_barrier_semaphore()
        for d in range(1, N_DEV):
            pl.semaphore_signal(
                barrier, inc=1,
                device_id=(lax.rem(my + d, N_DEV),),
                device_id_type=pl.DeviceIdType.MESH,
            )
        pl.semaphore_wait(barrier, N_DEV - 1)

        e = idx_ref[:, :]
        oh_m = e == lax.broadcasted_iota(jnp.int32, (N_TOK, N_EXP), 1)
        oh = oh_m.astype(jnp.bfloat16)
        r = lax.broadcasted_iota(jnp.int32, (N_TOK, N_TOK), 0)
        c = lax.broadcasted_iota(jnp.int32, (N_TOK, N_TOK), 1)
        tri = (r > c).astype(jnp.bfloat16)
        prior = jnp.dot(tri, oh, preferred_element_type=jnp.float32)
        pos = jnp.sum(jnp.where(oh_m, prior, 0.0), axis=1,
                      keepdims=True).astype(jnp.int32)
        keep = pos < CAP

        owner = lax.div(e, E_LOCAL)
        le = lax.rem(e, E_LOCAL)
        mine = jnp.logical_and(owner == my, keep)
        s_col = jnp.where(mine, le * CAP + pos, -1)
        st = (s_col == lax.broadcasted_iota(jnp.int32, (N_TOK, SLOTS), 1)
              ).astype(jnp.float32)
        xs = lax.dot_general(st, x_ref[:, :], (((0,), (0,)), ((), ())),
                             preferred_element_type=jnp.float32)
        y0 = jnp.dot(xs, w_ref[0], preferred_element_type=jnp.float32)
        y1 = jnp.dot(xs, w_ref[1], preferred_element_type=jnp.float32)
        row = lax.broadcasted_iota(jnp.int32, (SLOTS, D_OUT), 0)
        comm_ref[0, :, :] = jnp.where(row < CAP, y0, y1).astype(jnp.bfloat16)

        rdmas = []
        for d in range(1, N_DEV):
            rdma = pltpu.make_async_remote_copy(
                src_ref=comm_ref.at[0],
                dst_ref=comm_ref.at[d],
                send_sem=send_sems.at[d - 1],
                recv_sem=recv_sems.at[d - 1],
                device_id=(lax.rem(my + d, N_DEV),),
                device_id_type=pl.DeviceIdType.MESH,
            )
            rdma.start()
            rdmas.append(rdma)

        k = lax.rem(my - owner + N_DEV, N_DEV)
        g_col = jnp.where(keep, k * SLOTS + le * CAP + pos, -1)
        p = (g_col == lax.broadcasted_iota(jnp.int32, (N_TOK, N_SLOTS), 1)
             ).astype(jnp.bfloat16)

        for rdma in rdmas:
            rdma.wait_recv()

        g = comm_ref[:, :, :].reshape(N_SLOTS, D_OUT)
        out_ref[:, :] = jnp.dot(p, g, preferred_element_type=jnp.float32)

        for rdma in rdmas:
            rdma.wait_send()

    return pl.pallas_call(
        body,
        out_shape=jax.ShapeDtypeStruct((N_TOK, D_OUT), jnp.float32),
        in_specs=[pl.BlockSpec(memory_space=pltpu.VMEM)] * 3,
        out_specs=pl.BlockSpec(memory_space=pltpu.VMEM),
        scratch_shapes=[
            pltpu.VMEM((N_DEV, SLOTS, D_OUT), jnp.bfloat16),
            pltpu.SemaphoreType.DMA((N_DEV - 1,)),
            pltpu.SemaphoreType.DMA((N_DEV - 1,)),
        ],
        compiler_params=pltpu.CompilerParams(collective_id=0),
    )(x, route_idx, expert_W)


# device time: 20238 ns/iter; 3.8561x vs baseline; 1.0604x over previous
import jax
import jax.numpy as jnp
from jax import lax
from jax.experimental import pallas as pl
from jax.experimental.pallas import tpu as pltpu

N_DEV = 32
N_TOK = 512
N_EXP = 64
D_IN = 256
D_OUT = 512
E_LOCAL = 2
CAP = 6
SLOTS = E_LOCAL * CAP
N_SLOTS = N_DEV * SLOTS


def kernel(x, router_W, route_idx, expert_W):
    del router_W

    def body(x_ref, idx_ref, w_ref, out_ref, comm_ref, send_sems, recv_sems):
        my = lax.axis_index("i")

        barrier = pltpu.get_barrier_semaphore()
        for d in range(1, N_DEV):
            pl.semaphore_signal(
                barrier, inc=1,
                device_id=(lax.rem(my + d, N_DEV),),
                device_id_type=pl.DeviceIdType.MESH,
            )

        e = idx_ref[:, :]
        oh_m = e == lax.broadcasted_iota(jnp.int32, (N_TOK, N_EXP), 1)
        oh = oh_m.astype(jnp.bfloat16)
        r = lax.broadcasted_iota(jnp.int32, (N_TOK, N_TOK), 0)
        c = lax.broadcasted_iota(jnp.int32, (N_TOK, N_TOK), 1)
        tri = (r > c).astype(jnp.bfloat16)
        prior = jnp.dot(tri, oh, preferred_element_type=jnp.float32)
        pos = jnp.sum(jnp.where(oh_m, prior, 0.0), axis=1,
                      keepdims=True).astype(jnp.int32)
        keep = pos < CAP

        owner = lax.div(e, E_LOCAL)
        le = lax.rem(e, E_LOCAL)
        mine = jnp.logical_and(owner == my, keep)
        s_col = jnp.where(mine, le * CAP + pos, -1)
        st = (s_col == lax.broadcasted_iota(jnp.int32, (N_TOK, SLOTS), 1)
              ).astype(jnp.float32)
        xs = lax.dot_general(st, x_ref[:, :], (((0,), (0,)), ((), ())),
                             preferred_element_type=jnp.float32)
        y0 = jnp.dot(xs, w_ref[0], preferred_element_type=jnp.float32)
        y1 = jnp.dot(xs, w_ref[1], preferred_element_type=jnp.float32)
        row = lax.broadcasted_iota(jnp.int32, (SLOTS, D_OUT), 0)
        comm_ref[0, :, :] = jnp.where(row < CAP, y0, y1).astype(jnp.bfloat16)

        k = lax.rem(my - owner + N_DEV, N_DEV)
        g_col = jnp.where(keep, k * SLOTS + le * CAP + pos, -1)
        p = (g_col == lax.broadcasted_iota(jnp.int32, (N_TOK, N_SLOTS), 1)
             ).astype(jnp.bfloat16)

        pl.semaphore_wait(barrier, N_DEV - 1)

        rdmas = []
        for d in range(1, N_DEV):
            rdma = pltpu.make_async_remote_copy(
                src_ref=comm_ref.at[0],
                dst_ref=comm_ref.at[d],
                send_sem=send_sems.at[d - 1],
                recv_sem=recv_sems.at[d - 1],
                device_id=(lax.rem(my + d, N_DEV),),
                device_id_type=pl.DeviceIdType.MESH,
            )
            rdma.start()
            rdmas.append(rdma)

        for rdma in rdmas:
            rdma.wait_recv()

        g = comm_ref[:, :, :].reshape(N_SLOTS, D_OUT)
        out_ref[:, :] = jnp.dot(p, g, preferred_element_type=jnp.float32)

        for rdma in rdmas:
            rdma.wait_send()

    return pl.pallas_call(
        body,
        out_shape=jax.ShapeDtypeStruct((N_TOK, D_OUT), jnp.float32),
        in_specs=[pl.BlockSpec(memory_space=pltpu.VMEM)] * 3,
        out_specs=pl.BlockSpec(memory_space=pltpu.VMEM),
        scratch_shapes=[
            pltpu.VMEM((N_DEV, SLOTS, D_OUT), jnp.bfloat16),
            pltpu.SemaphoreType.DMA((N_DEV - 1,)),
            pltpu.SemaphoreType.DMA((N_DEV - 1,)),
        ],
        compiler_params=pltpu.CompilerParams(collective_id=0),
    )(x, route_idx, expert_W)
